# Initial kernel scaffold; baseline (speedup 1.0000x reference)
import jax
import jax.numpy as jnp
from jax.experimental import pallas as pl


def kernel(feat0, feat1, edge_index0, edge_index1, W_fc0, b_fc0, W_fc1, b_fc1, W_gcn0, b_gcn0, a0, W_gcn1, b_gcn1, a1, W_att, b_att, att_vec):
    raise NotImplementedError("write your pallas kernel here")


# speedup vs baseline: 3.2254x; 3.2254x over previous
import functools

import jax
import jax.numpy as jnp
from jax import lax
from jax.experimental import pallas as pl
from jax.experimental.pallas import tpu as pltpu
from jax.experimental.pallas import tpu_sc as plsc

N0 = 10000
E = 320000
H = 128

NC = 2
NS = 16

K = 128
CPT = 160
EP = NS * CPT * K
N0P = 10240
ROWS_PT = N0P // NS
ZR = 128
BLK = 16
PAD_DST = N0P - 1

_DN = (((1,), (1,)), ((), ()))


def _proj_body(feat_ref, wfc_ref, bfc_ref, wg0_ref, wg1_ref, seq0_ref, seq1_ref):
    x = feat_ref[...]
    h = lax.dot_general(x, wfc_ref[...], _DN, preferred_element_type=jnp.float32)
    h = h + bfc_ref[...]
    h = jnp.where(h > 0, h, jnp.exp(h) - 1.0)
    seq0_ref[...] = lax.dot_general(h, wg0_ref[...], _DN,
                                    preferred_element_type=jnp.float32)
    seq1_ref[...] = lax.dot_general(h, wg1_ref[...], _DN,
                                    preferred_element_type=jnp.float32)


def _project(feat0, W_fc0, b_fc0, W_gcn0, W_gcn1):
    R = 1000
    grid = (N0 // R,)
    return pl.pallas_call(
        _proj_body,
        grid=grid,
        in_specs=[
            pl.BlockSpec((R, feat0.shape[1]), lambda i: (i, 0)),
            pl.BlockSpec(W_fc0.shape, lambda i: (0, 0)),
            pl.BlockSpec((1, H), lambda i: (0, 0)),
            pl.BlockSpec((H, H), lambda i: (0, 0)),
            pl.BlockSpec((H, H), lambda i: (0, 0)),
        ],
        out_specs=[
            pl.BlockSpec((R, H), lambda i: (i, 0)),
            pl.BlockSpec((R, H), lambda i: (i, 0)),
        ],
        out_shape=[jax.ShapeDtypeStruct((N0, H), jnp.float32)] * 2,
    )(feat0, W_fc0, b_fc0.reshape(1, H), W_gcn0, W_gcn1)


def _seg_body(seq0_hbm, seq1_hbm, src0_hbm, dst0_hbm, src1_hbm, dst1_hbm,
              out0_hbm, out1_hbm, sidx, didx, rows, acc, sem):
    cid = lax.axis_index("c")
    tid = lax.axis_index("s")

    zero = jnp.zeros((16,), jnp.float32)

    def zrow(i, c):
        for j in range(H // 16):
            rows[i, pl.ds(j * 16, 16)] = zero
        return c

    lax.fori_loop(0, ZR, zrow, 0)
    for r in range(ROWS_PT // ZR):
        pltpu.sync_copy(rows, acc.at[pl.ds(tid * ROWS_PT + r * ZR, ZR)])
    plsc.subcore_barrier()

    def run(seq_hbm, src_hbm, dst_hbm, out_hbm):
        def blk(b, c):
            base = pl.multiple_of(tid * CPT + b * BLK, 8)
            pltpu.sync_copy(src_hbm.at[pl.ds(base, BLK)], sidx)
            pltpu.sync_copy(dst_hbm.at[pl.ds(base, BLK)], didx)

            def chunk(j, c2):
                pltpu.async_copy(seq_hbm.at[sidx.at[j]], rows, sem).wait()
                pltpu.sync_copy(rows, acc.at[didx.at[j]], add=True)
                return c2

            lax.fori_loop(0, BLK, chunk, 0)
            return c

        lax.fori_loop(0, CPT // BLK, blk, 0)
        plsc.subcore_barrier()
        sl = pl.ds(tid * ROWS_PT, ROWS_PT)
        pltpu.sync_copy(acc.at[sl], out_hbm.at[sl])

    @pl.when(cid == 0)
    def _():
        run(seq0_hbm, src0_hbm, dst0_hbm, out0_hbm)

    @pl.when(cid == 1)
    def _():
        run(seq1_hbm, src1_hbm, dst1_hbm, out1_hbm)


def _pad_idx(idx, fill):
    return jnp.concatenate(
        [idx, jnp.full((EP - E,), fill, jnp.int32)]).reshape(EP // K, K)


def _segment_sums(seq0, seq1, ei0, ei1):
    src0 = _pad_idx(ei0[0], 0)
    dst0 = _pad_idx(ei0[1], PAD_DST)
    src1 = _pad_idx(ei1[0], 0)
    dst1 = _pad_idx(ei1[1], PAD_DST)
    mesh = plsc.VectorSubcoreMesh(core_axis_name="c", subcore_axis_name="s")
    f = functools.partial(
        pl.kernel,
        out_type=[jax.ShapeDtypeStruct((N0P, H), jnp.float32)] * 2,
        mesh=mesh,
        scratch_types=[
            pltpu.VMEM((BLK, K), jnp.int32),
            pltpu.VMEM((BLK, K), jnp.int32),
            pltpu.VMEM((K, H), jnp.float32),
            pltpu.VMEM_SHARED((N0P, H), jnp.float32),
            pltpu.SemaphoreType.DMA,
        ],
    )(_seg_body)
    return f(seq0, seq1, src0, dst0, src1, dst1)


def _att_body(out0_ref, out1_ref, bg0_ref, bg1_ref, a0_ref, a1_ref,
              watt_ref, batt_ref, e0_ref, e1_ref, sp_ref):
    i = pl.program_id(0)
    x0 = out0_ref[...] + bg0_ref[...]
    e0 = jnp.where(x0 >= 0, x0, a0_ref[0, 0] * x0)
    e0_ref[...] = e0
    x1 = out1_ref[...] + bg1_ref[...]
    e1 = jnp.where(x1 >= 0, x1, a1_ref[0, 0] * x1)
    e1_ref[...] = e1
    t0 = jnp.tanh(lax.dot_general(e0, watt_ref[...], _DN,
                                  preferred_element_type=jnp.float32)
                  + batt_ref[...])
    t1 = jnp.tanh(lax.dot_general(e1, watt_ref[...], _DN,
                                  preferred_element_type=jnp.float32)
                  + batt_ref[...])

    @pl.when(i == 0)
    def _():
        sp_ref[...] = jnp.zeros_like(sp_ref)

    sp_ref[0:1, :] = sp_ref[0:1, :] + jnp.sum(t0, axis=0, keepdims=True)
    sp_ref[1:2, :] = sp_ref[1:2, :] + jnp.sum(t1, axis=0, keepdims=True)


def _attention(out0, out1, b_gcn0, a0, b_gcn1, a1, W_att, b_att):
    R = 1000
    grid = (N0 // R,)
    vec = lambda v: v.reshape(1, H)
    return pl.pallas_call(
        _att_body,
        grid=grid,
        in_specs=[
            pl.BlockSpec((R, H), lambda i: (i, 0)),
            pl.BlockSpec((R, H), lambda i: (i, 0)),
            pl.BlockSpec((1, H), lambda i: (0, 0)),
            pl.BlockSpec((1, H), lambda i: (0, 0)),
            pl.BlockSpec((1, 1), lambda i: (0, 0)),
            pl.BlockSpec((1, 1), lambda i: (0, 0)),
            pl.BlockSpec((H, H), lambda i: (0, 0)),
            pl.BlockSpec((1, H), lambda i: (0, 0)),
        ],
        out_specs=[
            pl.BlockSpec((R, H), lambda i: (i, 0)),
            pl.BlockSpec((R, H), lambda i: (i, 0)),
            pl.BlockSpec((8, H), lambda i: (0, 0)),
        ],
        out_shape=[
            jax.ShapeDtypeStruct((N0, H), jnp.float32),
            jax.ShapeDtypeStruct((N0, H), jnp.float32),
            jax.ShapeDtypeStruct((8, H), jnp.float32),
        ],
    )(out0, out1, vec(b_gcn0), vec(b_gcn1), a0.reshape(1, 1), a1.reshape(1, 1),
      W_att, vec(b_att))


def _comb_body(e0_ref, e1_ref, sp_ref, av_ref, z_ref):
    av = av_ref[...]
    s0 = jnp.sum(av * sp_ref[0:1, :]) / N0
    s1 = jnp.sum(av * sp_ref[1:2, :]) / N0
    m = jnp.maximum(s0, s1)
    w0 = jnp.exp(s0 - m)
    w1 = jnp.exp(s1 - m)
    t = w0 + w1
    z_ref[...] = e0_ref[...] * (w0 / t) + e1_ref[...] * (w1 / t)


def _combine(e0, e1, sp, att_vec):
    R = 1000
    grid = (N0 // R,)
    return pl.pallas_call(
        _comb_body,
        grid=grid,
        in_specs=[
            pl.BlockSpec((R, H), lambda i: (i, 0)),
            pl.BlockSpec((R, H), lambda i: (i, 0)),
            pl.BlockSpec((8, H), lambda i: (0, 0)),
            pl.BlockSpec((1, H), lambda i: (0, 0)),
        ],
        out_specs=pl.BlockSpec((R, H), lambda i: (i, 0)),
        out_shape=jax.ShapeDtypeStruct((N0, H), jnp.float32),
    )(e0, e1, sp, att_vec)


def kernel(feat0, feat1, edge_index0, edge_index1, W_fc0, b_fc0, W_fc1, b_fc1,
           W_gcn0, b_gcn0, a0, W_gcn1, b_gcn1, a1, W_att, b_att, att_vec):
    seq0, seq1 = _project(feat0, W_fc0, b_fc0, W_gcn0, W_gcn1)
    out0, out1 = _segment_sums(seq0, seq1, edge_index0, edge_index1)
    e0, e1, sp = _attention(out0, out1, b_gcn0, a0, b_gcn1, a1, W_att, b_att)
    return _combine(e0, e1, sp, att_vec)


# speedup vs baseline: 3.8630x; 1.1977x over previous
import functools

import jax
import jax.numpy as jnp
from jax import lax
from jax.experimental import pallas as pl
from jax.experimental.pallas import tpu as pltpu
from jax.experimental.pallas import tpu_sc as plsc

N0 = 10000
E = 320000
H = 128

NC = 2
NS = 16

K = 128
CPT = 160
EP = NS * CPT * K
N0P = 10240
ROWS_PT = N0P // NS
ZR = 128
BLK = 16
PAD_DST = N0P - 1

_DN = (((1,), (1,)), ((), ()))


def _proj_body(feat_ref, wfc_ref, bfc_ref, wg0_ref, wg1_ref, seq0_ref, seq1_ref):
    x = feat_ref[...]
    h = lax.dot_general(x, wfc_ref[...], _DN, preferred_element_type=jnp.float32)
    h = h + bfc_ref[...]
    h = jnp.where(h > 0, h, jnp.exp(h) - 1.0)
    seq0_ref[...] = lax.dot_general(h, wg0_ref[...], _DN,
                                    preferred_element_type=jnp.float32)
    seq1_ref[...] = lax.dot_general(h, wg1_ref[...], _DN,
                                    preferred_element_type=jnp.float32)


def _project(feat0, W_fc0, b_fc0, W_gcn0, W_gcn1):
    R = 1000
    grid = (N0 // R,)
    return pl.pallas_call(
        _proj_body,
        grid=grid,
        in_specs=[
            pl.BlockSpec((R, feat0.shape[1]), lambda i: (i, 0)),
            pl.BlockSpec(W_fc0.shape, lambda i: (0, 0)),
            pl.BlockSpec((1, H), lambda i: (0, 0)),
            pl.BlockSpec((H, H), lambda i: (0, 0)),
            pl.BlockSpec((H, H), lambda i: (0, 0)),
        ],
        out_specs=[
            pl.BlockSpec((R, H), lambda i: (i, 0)),
            pl.BlockSpec((R, H), lambda i: (i, 0)),
        ],
        out_shape=[jax.ShapeDtypeStruct((N0, H), jnp.float32)] * 2,
    )(feat0, W_fc0, b_fc0.reshape(1, H), W_gcn0, W_gcn1)


def _seg_body(seq0_hbm, seq1_hbm, src0_hbm, dst0_hbm, src1_hbm, dst1_hbm,
              out0_hbm, out1_hbm, sidx, didx, rows0, rows1, acc,
              isem, g0, g1):
    cid = lax.axis_index("c")
    tid = lax.axis_index("s")
    NB = CPT // BLK

    zero = jnp.zeros((16,), jnp.float32)

    def zrow(i, c):
        for j in range(H // 16):
            rows0[i, pl.ds(j * 16, 16)] = zero
        return c

    lax.fori_loop(0, ZR, zrow, 0)
    for r in range(ROWS_PT // ZR):
        pltpu.sync_copy(rows0, acc.at[pl.ds(tid * ROWS_PT + r * ZR, ZR)])
    plsc.subcore_barrier()

    def run(seq_hbm, src_hbm, dst_hbm, out_hbm):
        def idx_base(b):
            return pl.multiple_of(tid * CPT + b * BLK, 8)

        pltpu.sync_copy(src_hbm.at[pl.ds(idx_base(0), BLK)], sidx.at[0])
        pltpu.sync_copy(dst_hbm.at[pl.ds(idx_base(0), BLK)], didx.at[0])
        cg0 = pltpu.async_copy(seq_hbm.at[sidx.at[0].at[0]], rows0, g0)

        for b in range(NB):
            s = b % 2
            ss, ds_ = sidx.at[s], didx.at[s]
            if b + 1 < NB:
                ci0 = pltpu.async_copy(
                    src_hbm.at[pl.ds(idx_base(b + 1), BLK)], sidx.at[1 - s],
                    isem)
                ci1 = pltpu.async_copy(
                    dst_hbm.at[pl.ds(idx_base(b + 1), BLK)], didx.at[1 - s],
                    isem)

            def pair(p, c):
                pltpu.async_copy(seq_hbm.at[ss.at[2 * p + 1]], rows1, g1)
                pltpu.make_async_copy(seq_hbm.at[ss.at[2 * p]], rows0,
                                      g0).wait()
                pltpu.sync_copy(rows0, acc.at[ds_.at[2 * p]], add=True)
                pltpu.async_copy(seq_hbm.at[ss.at[2 * p + 2]], rows0, g0)
                pltpu.make_async_copy(seq_hbm.at[ss.at[2 * p + 1]], rows1,
                                      g1).wait()
                pltpu.sync_copy(rows1, acc.at[ds_.at[2 * p + 1]], add=True)
                return c

            lax.fori_loop(0, BLK // 2 - 1, pair, 0)

            jt = BLK - 2
            pltpu.async_copy(seq_hbm.at[ss.at[jt + 1]], rows1, g1)
            cg0 = pltpu.make_async_copy(seq_hbm.at[ss.at[jt]], rows0, g0)
            cg0.wait()
            pltpu.sync_copy(rows0, acc.at[ds_.at[jt]], add=True)
            if b + 1 < NB:
                ci0.wait()
                ci1.wait()
                pltpu.async_copy(seq_hbm.at[sidx.at[1 - s].at[0]], rows0, g0)
            pltpu.make_async_copy(seq_hbm.at[ss.at[jt + 1]], rows1, g1).wait()
            pltpu.sync_copy(rows1, acc.at[ds_.at[jt + 1]], add=True)

        plsc.subcore_barrier()
        sl = pl.ds(tid * ROWS_PT, ROWS_PT)
        pltpu.sync_copy(acc.at[sl], out_hbm.at[sl])

    @pl.when(cid == 0)
    def _():
        run(seq0_hbm, src0_hbm, dst0_hbm, out0_hbm)

    @pl.when(cid == 1)
    def _():
        run(seq1_hbm, src1_hbm, dst1_hbm, out1_hbm)


def _pad_idx(idx, fill):
    return jnp.concatenate(
        [idx, jnp.full((EP - E,), fill, jnp.int32)]).reshape(EP // K, K)


def _segment_sums(seq0, seq1, ei0, ei1):
    src0 = _pad_idx(ei0[0], 0)
    dst0 = _pad_idx(ei0[1], PAD_DST)
    src1 = _pad_idx(ei1[0], 0)
    dst1 = _pad_idx(ei1[1], PAD_DST)
    mesh = plsc.VectorSubcoreMesh(core_axis_name="c", subcore_axis_name="s")
    f = functools.partial(
        pl.kernel,
        out_type=[jax.ShapeDtypeStruct((N0P, H), jnp.float32)] * 2,
        mesh=mesh,
        scratch_types=[
            pltpu.VMEM((2, BLK, K), jnp.int32),
            pltpu.VMEM((2, BLK, K), jnp.int32),
            pltpu.VMEM((K, H), jnp.float32),
            pltpu.VMEM((K, H), jnp.float32),
            pltpu.VMEM_SHARED((N0P, H), jnp.float32),
            pltpu.SemaphoreType.DMA,
            pltpu.SemaphoreType.DMA,
            pltpu.SemaphoreType.DMA,
        ],
    )(_seg_body)
    return f(seq0, seq1, src0, dst0, src1, dst1)


def _att_body(out0_ref, out1_ref, bg0_ref, bg1_ref, a0_ref, a1_ref,
              watt_ref, batt_ref, e0_ref, e1_ref, sp_ref):
    i = pl.program_id(0)
    x0 = out0_ref[...] + bg0_ref[...]
    e0 = jnp.where(x0 >= 0, x0, a0_ref[0, 0] * x0)
    e0_ref[...] = e0
    x1 = out1_ref[...] + bg1_ref[...]
    e1 = jnp.where(x1 >= 0, x1, a1_ref[0, 0] * x1)
    e1_ref[...] = e1
    t0 = jnp.tanh(lax.dot_general(e0, watt_ref[...], _DN,
                                  preferred_element_type=jnp.float32)
                  + batt_ref[...])
    t1 = jnp.tanh(lax.dot_general(e1, watt_ref[...], _DN,
                                  preferred_element_type=jnp.float32)
                  + batt_ref[...])

    @pl.when(i == 0)
    def _():
        sp_ref[...] = jnp.zeros_like(sp_ref)

    sp_ref[0:1, :] = sp_ref[0:1, :] + jnp.sum(t0, axis=0, keepdims=True)
    sp_ref[1:2, :] = sp_ref[1:2, :] + jnp.sum(t1, axis=0, keepdims=True)


def _attention(out0, out1, b_gcn0, a0, b_gcn1, a1, W_att, b_att):
    R = 1000
    grid = (N0 // R,)
    vec = lambda v: v.reshape(1, H)
    return pl.pallas_call(
        _att_body,
        grid=grid,
        in_specs=[
            pl.BlockSpec((R, H), lambda i: (i, 0)),
            pl.BlockSpec((R, H), lambda i: (i, 0)),
            pl.BlockSpec((1, H), lambda i: (0, 0)),
            pl.BlockSpec((1, H), lambda i: (0, 0)),
            pl.BlockSpec((1, 1), lambda i: (0, 0)),
            pl.BlockSpec((1, 1), lambda i: (0, 0)),
            pl.BlockSpec((H, H), lambda i: (0, 0)),
            pl.BlockSpec((1, H), lambda i: (0, 0)),
        ],
        out_specs=[
            pl.BlockSpec((R, H), lambda i: (i, 0)),
            pl.BlockSpec((R, H), lambda i: (i, 0)),
            pl.BlockSpec((8, H), lambda i: (0, 0)),
        ],
        out_shape=[
            jax.ShapeDtypeStruct((N0, H), jnp.float32),
            jax.ShapeDtypeStruct((N0, H), jnp.float32),
            jax.ShapeDtypeStruct((8, H), jnp.float32),
        ],
    )(out0, out1, vec(b_gcn0), vec(b_gcn1), a0.reshape(1, 1), a1.reshape(1, 1),
      W_att, vec(b_att))


def _comb_body(e0_ref, e1_ref, sp_ref, av_ref, z_ref):
    av = av_ref[...]
    s0 = jnp.sum(av * sp_ref[0:1, :]) / N0
    s1 = jnp.sum(av * sp_ref[1:2, :]) / N0
    m = jnp.maximum(s0, s1)
    w0 = jnp.exp(s0 - m)
    w1 = jnp.exp(s1 - m)
    t = w0 + w1
    z_ref[...] = e0_ref[...] * (w0 / t) + e1_ref[...] * (w1 / t)


def _combine(e0, e1, sp, att_vec):
    R = 1000
    grid = (N0 // R,)
    return pl.pallas_call(
        _comb_body,
        grid=grid,
        in_specs=[
            pl.BlockSpec((R, H), lambda i: (i, 0)),
            pl.BlockSpec((R, H), lambda i: (i, 0)),
            pl.BlockSpec((8, H), lambda i: (0, 0)),
            pl.BlockSpec((1, H), lambda i: (0, 0)),
        ],
        out_specs=pl.BlockSpec((R, H), lambda i: (i, 0)),
        out_shape=jax.ShapeDtypeStruct((N0, H), jnp.float32),
    )(e0, e1, sp, att_vec)


def kernel(feat0, feat1, edge_index0, edge_index1, W_fc0, b_fc0, W_fc1, b_fc1,
           W_gcn0, b_gcn0, a0, W_gcn1, b_gcn1, a1, W_att, b_att, att_vec):
    seq0, seq1 = _project(feat0, W_fc0, b_fc0, W_gcn0, W_gcn1)
    out0, out1 = _segment_sums(seq0, seq1, edge_index0, edge_index1)
    e0, e1, sp = _attention(out0, out1, b_gcn0, a0, b_gcn1, a1, W_att, b_att)
    return _combine(e0, e1, sp, att_vec)


# speedup vs baseline: 4.2238x; 1.0934x over previous
import functools

import jax
import jax.numpy as jnp
from jax import lax
from jax.experimental import pallas as pl
from jax.experimental.pallas import tpu as pltpu
from jax.experimental.pallas import tpu_sc as plsc

N0 = 10000
E = 320000
H = 128

NC = 2
NS = 16

K = 128
CPT = 160
EP = NS * CPT * K
N0P = 10240
ROWS_PT = N0P // NS
ZR = 128
BLK = 16
PAD_DST = N0P - 1

_DN = (((1,), (1,)), ((), ()))


def _proj_body(feat_ref, wfc_ref, bfc_ref, wg0_ref, wg1_ref, seq0_ref, seq1_ref):
    x = feat_ref[...]
    h = lax.dot_general(x, wfc_ref[...], _DN, preferred_element_type=jnp.float32)
    h = h + bfc_ref[...]
    h = jnp.where(h > 0, h, jnp.exp(h) - 1.0)
    seq0_ref[...] = lax.dot_general(h, wg0_ref[...], _DN,
                                    preferred_element_type=jnp.float32)
    seq1_ref[...] = lax.dot_general(h, wg1_ref[...], _DN,
                                    preferred_element_type=jnp.float32)


def _project(feat0, W_fc0, b_fc0, W_gcn0, W_gcn1):
    R = 2000
    grid = (N0 // R,)
    return pl.pallas_call(
        _proj_body,
        grid=grid,
        in_specs=[
            pl.BlockSpec((R, feat0.shape[1]), lambda i: (i, 0)),
            pl.BlockSpec(W_fc0.shape, lambda i: (0, 0)),
            pl.BlockSpec((1, H), lambda i: (0, 0)),
            pl.BlockSpec((H, H), lambda i: (0, 0)),
            pl.BlockSpec((H, H), lambda i: (0, 0)),
        ],
        out_specs=[
            pl.BlockSpec((R, H), lambda i: (i, 0)),
            pl.BlockSpec((R, H), lambda i: (i, 0)),
        ],
        out_shape=[jax.ShapeDtypeStruct((N0, H), jnp.float32)] * 2,
    )(feat0, W_fc0, b_fc0.reshape(1, H), W_gcn0, W_gcn1)


def _seg_body(seq0_hbm, seq1_hbm, src0_hbm, dst0_hbm, src1_hbm, dst1_hbm,
              out0_hbm, out1_hbm, sidx, didx, rows0, rows1, acc,
              isem, g0, g1, sa, sb):
    cid = lax.axis_index("c")
    tid = lax.axis_index("s")
    NB = CPT // BLK

    zero = jnp.zeros((16,), jnp.float32)

    def zrow(i, c):
        for j in range(H // 16):
            rows0[i, pl.ds(j * 16, 16)] = zero
        return c

    lax.fori_loop(0, ZR, zrow, 0)
    for r in range(ROWS_PT // ZR):
        pltpu.sync_copy(rows0, acc.at[pl.ds(tid * ROWS_PT + r * ZR, ZR)])
    plsc.subcore_barrier()

    def run(seq_hbm, src_hbm, dst_hbm, out_hbm):
        def idx_base(b):
            return pl.multiple_of(tid * CPT + b * BLK, 8)

        def scat(rows, dref, j):
            ca = pltpu.async_copy(rows.at[pl.ds(0, K // 2)],
                                  acc.at[dref.at[2 * j]], sa, add=True)
            cb = pltpu.async_copy(rows.at[pl.ds(K // 2, K // 2)],
                                  acc.at[dref.at[2 * j + 1]], sb, add=True)
            ca.wait()
            cb.wait()

        pltpu.sync_copy(src_hbm.at[pl.ds(idx_base(0), BLK)], sidx.at[0])
        pltpu.sync_copy(
            dst_hbm.at[pl.ds(pl.multiple_of(2 * idx_base(0), 16), 2 * BLK)],
            didx.at[0])
        cg0 = pltpu.async_copy(seq_hbm.at[sidx.at[0].at[0]], rows0, g0)

        for b in range(NB):
            s = b % 2
            ss, ds_ = sidx.at[s], didx.at[s]
            if b + 1 < NB:
                ci0 = pltpu.async_copy(
                    src_hbm.at[pl.ds(idx_base(b + 1), BLK)], sidx.at[1 - s],
                    isem)
                ci1 = pltpu.async_copy(
                    dst_hbm.at[pl.ds(pl.multiple_of(2 * idx_base(b + 1), 16),
                                     2 * BLK)],
                    didx.at[1 - s], isem)

            def pair(p, c):
                pltpu.async_copy(seq_hbm.at[ss.at[2 * p + 1]], rows1, g1)
                pltpu.make_async_copy(seq_hbm.at[ss.at[2 * p]], rows0,
                                      g0).wait()
                scat(rows0, ds_, 2 * p)
                pltpu.async_copy(seq_hbm.at[ss.at[2 * p + 2]], rows0, g0)
                pltpu.make_async_copy(seq_hbm.at[ss.at[2 * p + 1]], rows1,
                                      g1).wait()
                scat(rows1, ds_, 2 * p + 1)
                return c

            lax.fori_loop(0, BLK // 2 - 1, pair, 0)

            jt = BLK - 2
            pltpu.async_copy(seq_hbm.at[ss.at[jt + 1]], rows1, g1)
            cg0 = pltpu.make_async_copy(seq_hbm.at[ss.at[jt]], rows0, g0)
            cg0.wait()
            scat(rows0, ds_, jt)
            if b + 1 < NB:
                ci0.wait()
                ci1.wait()
                pltpu.async_copy(seq_hbm.at[sidx.at[1 - s].at[0]], rows0, g0)
            pltpu.make_async_copy(seq_hbm.at[ss.at[jt + 1]], rows1, g1).wait()
            scat(rows1, ds_, jt + 1)

        plsc.subcore_barrier()
        sl = pl.ds(tid * ROWS_PT, ROWS_PT)
        pltpu.sync_copy(acc.at[sl], out_hbm.at[sl])

    @pl.when(cid == 0)
    def _():
        run(seq0_hbm, src0_hbm, dst0_hbm, out0_hbm)

    @pl.when(cid == 1)
    def _():
        run(seq1_hbm, src1_hbm, dst1_hbm, out1_hbm)


def _pad_idx(idx, fill, k):
    return jnp.concatenate(
        [idx, jnp.full((EP - E,), fill, jnp.int32)]).reshape(EP // k, k)


def _segment_sums(seq0, seq1, ei0, ei1):
    src0 = _pad_idx(ei0[0], 0, K)
    dst0 = _pad_idx(ei0[1], PAD_DST, K // 2)
    src1 = _pad_idx(ei1[0], 0, K)
    dst1 = _pad_idx(ei1[1], PAD_DST, K // 2)
    mesh = plsc.VectorSubcoreMesh(core_axis_name="c", subcore_axis_name="s")
    f = functools.partial(
        pl.kernel,
        out_type=[jax.ShapeDtypeStruct((N0P, H), jnp.float32)] * 2,
        mesh=mesh,
        scratch_types=[
            pltpu.VMEM((2, BLK, K), jnp.int32),
            pltpu.VMEM((2, 2 * BLK, K // 2), jnp.int32),
            pltpu.VMEM((K, H), jnp.float32),
            pltpu.VMEM((K, H), jnp.float32),
            pltpu.VMEM_SHARED((N0P, H), jnp.float32),
            pltpu.SemaphoreType.DMA,
            pltpu.SemaphoreType.DMA,
            pltpu.SemaphoreType.DMA,
            pltpu.SemaphoreType.DMA,
            pltpu.SemaphoreType.DMA,
        ],
    )(_seg_body)
    return f(seq0, seq1, src0, dst0, src1, dst1)


def _att_body(out0_ref, out1_ref, bg0_ref, bg1_ref, a0_ref, a1_ref,
              watt_ref, batt_ref, e0_ref, e1_ref, sp_ref):
    i = pl.program_id(0)
    x0 = out0_ref[...] + bg0_ref[...]
    e0 = jnp.where(x0 >= 0, x0, a0_ref[0, 0] * x0)
    e0_ref[...] = e0
    x1 = out1_ref[...] + bg1_ref[...]
    e1 = jnp.where(x1 >= 0, x1, a1_ref[0, 0] * x1)
    e1_ref[...] = e1
    t0 = jnp.tanh(lax.dot_general(e0, watt_ref[...], _DN,
                                  preferred_element_type=jnp.float32)
                  + batt_ref[...])
    t1 = jnp.tanh(lax.dot_general(e1, watt_ref[...], _DN,
                                  preferred_element_type=jnp.float32)
                  + batt_ref[...])

    @pl.when(i == 0)
    def _():
        sp_ref[...] = jnp.zeros_like(sp_ref)

    sp_ref[0:1, :] = sp_ref[0:1, :] + jnp.sum(t0, axis=0, keepdims=True)
    sp_ref[1:2, :] = sp_ref[1:2, :] + jnp.sum(t1, axis=0, keepdims=True)


def _attention(out0, out1, b_gcn0, a0, b_gcn1, a1, W_att, b_att):
    R = 2000
    grid = (N0 // R,)
    vec = lambda v: v.reshape(1, H)
    return pl.pallas_call(
        _att_body,
        grid=grid,
        in_specs=[
            pl.BlockSpec((R, H), lambda i: (i, 0)),
            pl.BlockSpec((R, H), lambda i: (i, 0)),
            pl.BlockSpec((1, H), lambda i: (0, 0)),
            pl.BlockSpec((1, H), lambda i: (0, 0)),
            pl.BlockSpec((1, 1), lambda i: (0, 0)),
            pl.BlockSpec((1, 1), lambda i: (0, 0)),
            pl.BlockSpec((H, H), lambda i: (0, 0)),
            pl.BlockSpec((1, H), lambda i: (0, 0)),
        ],
        out_specs=[
            pl.BlockSpec((R, H), lambda i: (i, 0)),
            pl.BlockSpec((R, H), lambda i: (i, 0)),
            pl.BlockSpec((8, H), lambda i: (0, 0)),
        ],
        out_shape=[
            jax.ShapeDtypeStruct((N0, H), jnp.float32),
            jax.ShapeDtypeStruct((N0, H), jnp.float32),
            jax.ShapeDtypeStruct((8, H), jnp.float32),
        ],
    )(out0, out1, vec(b_gcn0), vec(b_gcn1), a0.reshape(1, 1), a1.reshape(1, 1),
      W_att, vec(b_att))


def _comb_body(e0_ref, e1_ref, sp_ref, av_ref, z_ref):
    av = av_ref[...]
    s0 = jnp.sum(av * sp_ref[0:1, :]) / N0
    s1 = jnp.sum(av * sp_ref[1:2, :]) / N0
    m = jnp.maximum(s0, s1)
    w0 = jnp.exp(s0 - m)
    w1 = jnp.exp(s1 - m)
    t = w0 + w1
    z_ref[...] = e0_ref[...] * (w0 / t) + e1_ref[...] * (w1 / t)


def _combine(e0, e1, sp, att_vec):
    R = 1000
    grid = (N0 // R,)
    return pl.pallas_call(
        _comb_body,
        grid=grid,
        in_specs=[
            pl.BlockSpec((R, H), lambda i: (i, 0)),
            pl.BlockSpec((R, H), lambda i: (i, 0)),
            pl.BlockSpec((8, H), lambda i: (0, 0)),
            pl.BlockSpec((1, H), lambda i: (0, 0)),
        ],
        out_specs=pl.BlockSpec((R, H), lambda i: (i, 0)),
        out_shape=jax.ShapeDtypeStruct((N0, H), jnp.float32),
    )(e0, e1, sp, att_vec)


def kernel(feat0, feat1, edge_index0, edge_index1, W_fc0, b_fc0, W_fc1, b_fc1,
           W_gcn0, b_gcn0, a0, W_gcn1, b_gcn1, a1, W_att, b_att, att_vec):
    seq0, seq1 = _project(feat0, W_fc0, b_fc0, W_gcn0, W_gcn1)
    out0, out1 = _segment_sums(seq0, seq1, edge_index0, edge_index1)
    e0, e1, sp = _attention(out0, out1, b_gcn0, a0, b_gcn1, a1, W_att, b_att)
    return _combine(e0, e1, sp, att_vec)
